# baseline (speedup 1.0000x reference)
import jax
import jax.numpy as jnp
from jax import lax
from jax.experimental import pallas as pl
from jax.experimental.pallas import tpu as pltpu
from jax.experimental.pallas import tpu_sc as plsc

N_NODES = 100000
N_EDGES = 1600000
CW = 16
GRP = 125
GPS = 5
ROWS = N_EDGES // GRP
NC, NS = 2, 16
ROWS_PER_SC = ROWS // NC
ROWS_PER_TILE = ROWS_PER_SC // NS
STEPS = ROWS_PER_TILE // GPS
NPT = N_NODES // 10


def _make_sc_agg(nchunks):
  edge_split = nchunks == 1
  steps = STEPS if edge_split else 2 * STEPS
  rows_per_tile = ROWS_PER_TILE if edge_split else ROWS // NS

  def body(*refs):
    feats = refs[:nchunks]
    idx3, zero = refs[nchunks:nchunks + 2]
    outs = refs[nchunks + 2:2 * nchunks + 2]
    (acc, srcv0, dstv0, rows0, sem0,
     srcv1, dstv1, rows1, sem1) = refs[2 * nchunks + 2:]

    c = lax.axis_index("c")
    s = lax.axis_index("s")
    if edge_split:
      base = (c * NS + s) * rows_per_tile
    else:
      base = s * rows_per_tile

    def load_idx(r0, srcv, dstv):
      pltpu.sync_copy(idx3.at[0, pl.ds(r0, GPS)], srcv)
      pltpu.sync_copy(idx3.at[1, pl.ds(r0, GPS)], dstv)

    def chunk_pass(k):
      feat = feats[k]
      out = outs[k]

      def fire(srcv, rows, sem):
        for j in range(GPS):
          pltpu.async_copy(feat.at[srcv.at[j]], rows.at[j], sem)

      def drain(srcv, rows, sem):
        for j in range(GPS):
          pltpu.make_async_copy(feat.at[srcv.at[j]], rows.at[j], sem).wait()

      def scatter(dstv, rows):
        for j in range(GPS):
          pltpu.sync_copy(rows.at[j], acc.at[dstv.at[j]], add=True)

      @pl.when(s < 10)
      def _():
        pltpu.sync_copy(zero, acc.at[pl.ds(pl.multiple_of(s * NPT, 8), NPT)])
      plsc.subcore_barrier()

      load_idx(base, srcv0, dstv0)
      fire(srcv0, rows0, sem0)

      def pair(t, _):
        r0 = base + 2 * t * GPS
        load_idx(r0 + GPS, srcv1, dstv1)
        fire(srcv1, rows1, sem1)
        drain(srcv0, rows0, sem0)
        scatter(dstv0, rows0)
        load_idx(r0 + 2 * GPS, srcv0, dstv0)
        fire(srcv0, rows0, sem0)
        drain(srcv1, rows1, sem1)
        scatter(dstv1, rows1)
        return _

      lax.fori_loop(0, steps // 2 - 1, pair, 0)
      load_idx(base + (steps - 1) * GPS, srcv1, dstv1)
      fire(srcv1, rows1, sem1)
      drain(srcv0, rows0, sem0)
      scatter(dstv0, rows0)
      drain(srcv1, rows1, sem1)
      scatter(dstv1, rows1)
      plsc.subcore_barrier()

      @pl.when(s < 10)
      def _():
        row0 = pl.multiple_of(s * NPT, 8)
        if edge_split:
          dst = out.at[c, pl.ds(row0, NPT)]
        else:
          dst = out.at[pl.ds(row0, NPT)]
        pltpu.sync_copy(acc.at[pl.ds(row0, NPT)], dst)
      plsc.subcore_barrier()

    if edge_split:
      chunk_pass(0)
    else:
      @pl.when(c == 0)
      def _():
        chunk_pass(0)
        chunk_pass(1)

      @pl.when(c == 1)
      def _():
        chunk_pass(2)
        chunk_pass(3)

  out_shape = ((NC, N_NODES, CW) if edge_split else (N_NODES, CW))
  mesh = plsc.VectorSubcoreMesh(core_axis_name="c", subcore_axis_name="s")
  return pl.kernel(
      body,
      out_type=[jax.ShapeDtypeStruct(out_shape, jnp.float32)
                for _ in range(nchunks)],
      mesh=mesh,
      compiler_params=pltpu.CompilerParams(use_tc_tiling_on_sc=False),
      scratch_types=[
          pltpu.VMEM_SHARED((N_NODES, CW), jnp.float32),
          pltpu.VMEM((GPS, GRP), jnp.int32),
          pltpu.VMEM((GPS, GRP), jnp.int32),
          pltpu.VMEM((GPS, GRP, CW), jnp.float32),
          pltpu.SemaphoreType.DMA,
          pltpu.VMEM((GPS, GRP), jnp.int32),
          pltpu.VMEM((GPS, GRP), jnp.int32),
          pltpu.VMEM((GPS, GRP, CW), jnp.float32),
          pltpu.SemaphoreType.DMA,
      ],
  )


_sc_agg1 = _make_sc_agg(1)
_sc_agg4 = _make_sc_agg(4)

NP_ROWS = N_NODES // 8
TCG = 10
TNP = NP_ROWS // TCG


def _bd(W):
  W4 = W.reshape(4, CW, 4, CW)
  return jnp.einsum('kfpg,ab->kafpbg', W4, jnp.eye(8, dtype=W.dtype)
                    ).reshape(512, 512)


def _bd_in(W):
  W3 = W.reshape(CW, 4, CW)
  return jnp.einsum('fpg,ab->afpbg', W3, jnp.eye(8, dtype=W.dtype)
                    ).reshape(128, 512)


def _bd_out(W):
  out = W.shape[1]
  W3 = W.reshape(4, CW, out)
  return jnp.einsum('pgo,ba->pbgao', W3, jnp.eye(8, dtype=W.dtype)
                    ).reshape(512, 8 * out)


def _bvec(bv):
  return jnp.tile(bv.reshape(4, 1, CW), (1, 8, 1)).reshape(1, 512)


def _mlp(h, W1, b1, W2, b2):
  t = jnp.maximum(jnp.dot(h, W1, preferred_element_type=jnp.float32) + b1, 0.0)
  return jnp.dot(t, W2, preferred_element_type=jnp.float32) + b2


def _split_out(y, outs):
  for k, o in enumerate(outs):
    o[...] = y[:, 128 * k:128 * (k + 1)].reshape(1, TNP, 128)


def _tc_mlp1(xp_ref, p_ref, W1_ref, b1_ref, W2_ref, b2_ref, o0, o1, o2, o3):
  h = xp_ref[0] + p_ref[0, 0] + p_ref[1, 0]
  y = _mlp(h, W1_ref[...], b1_ref[...], W2_ref[...], b2_ref[...])
  _split_out(y, (o0, o1, o2, o3))


def _tc_mlp(h0, h1, h2, h3, p0, p1, p2, p3,
            W1_ref, b1_ref, W2_ref, b2_ref, o0, o1, o2, o3):
  h = jnp.concatenate([h0[0] + p0[0], h1[0] + p1[0],
                       h2[0] + p2[0], h3[0] + p3[0]], axis=1)
  y = _mlp(h, W1_ref[...], b1_ref[...], W2_ref[...], b2_ref[...])
  _split_out(y, (o0, o1, o2, o3))


def _tc_final(h0, h1, h2, h3, p0, p1, p2, p3,
              W1_ref, b1_ref, W2_ref, b2_ref,
              fW1_ref, fb1_ref, fW2_ref, fb2_ref, out_ref):
  h = jnp.concatenate([h0[0] + p0[0], h1[0] + p1[0],
                       h2[0] + p2[0], h3[0] + p3[0]], axis=1)
  y = _mlp(h, W1_ref[...], b1_ref[...], W2_ref[...], b2_ref[...])
  t = jnp.maximum(
      jnp.dot(y, fW1_ref[...], preferred_element_type=jnp.float32)
      + fb1_ref[...], 0.0)
  o = (jnp.dot(t, fW2_ref[...], preferred_element_type=jnp.float32)
       + fb2_ref[...])
  out_ref[...] = o.reshape(1, TNP, 8 * 28)


def _packed_spec():
  return pl.BlockSpec((1, TNP, 128), lambda i: (i, 0, 0))


def _partial_spec():
  return pl.BlockSpec((NC, 1, TNP, 128), lambda i: (0, i, 0, 0))


def _full_spec(shape):
  nd = len(shape)
  return pl.BlockSpec(shape, lambda i, _nd=nd: (0,) * _nd)


@jax.jit
def kernel(x, edge_index, c1_W1, c1_b1, c1_W2, c1_b2, c2_W1, c2_b1, c2_W2,
           c2_b2, c3_W1, c3_b1, c3_W2, c3_b2, f_W1, f_b1, f_W2, f_b2):
  grid = TCG
  f32 = jnp.float32

  xpP = jnp.pad(x.reshape(NP_ROWS, 8, x.shape[1]),
                ((0, 0), (0, 0), (0, CW - x.shape[1]))).reshape(
                    TCG, TNP, 128)
  xp = xpP.reshape(N_NODES, CW)
  idx3 = edge_index.reshape(2, ROWS, GRP)
  zero = jnp.zeros((NPT, CW), f32)
  W1p = jnp.pad(c1_W1, ((0, CW - c1_W1.shape[0]), (0, 0)))
  bd = {"c1_W1": _bd_in(W1p), "c1_W2": _bd(c1_W2),
        "c2_W1": _bd(c2_W1), "c2_W2": _bd(c2_W2),
        "c3_W1": _bd(c3_W1), "c3_W2": _bd(c3_W2),
        "f_W1": _bd(f_W1), "f_W2": _bd_out(f_W2)}
  b = {n: _bvec(v) for n, v in
       dict(c1_b1=c1_b1, c1_b2=c1_b2, c2_b1=c2_b1, c2_b2=c2_b2,
            c3_b1=c3_b1, c3_b2=c3_b2, f_b1=f_b1).items()}
  b["f_b2"] = jnp.tile(f_b2, (8,)).reshape(1, 8 * 28)

  (p1,) = _sc_agg1(xp, idx3, zero)
  h1 = pl.pallas_call(
      _tc_mlp1,
      grid=(grid,),
      in_specs=[_packed_spec(), _partial_spec(),
                _full_spec((128, 512)), _full_spec((1, 512)),
                _full_spec((512, 512)), _full_spec((1, 512))],
      out_specs=[_packed_spec()] * 4,
      out_shape=[jax.ShapeDtypeStruct((TCG, TNP, 128), f32)] * 4,
  )(xpP, p1.reshape(NC, TCG, TNP, 128), bd["c1_W1"], b["c1_b1"],
    bd["c1_W2"], b["c1_b2"])

  def gin_layer(h, W1, b1, W2, b2):
    ps = _sc_agg4(*[hk.reshape(N_NODES, CW) for hk in h], idx3, zero)
    return pl.pallas_call(
        _tc_mlp,
        grid=(grid,),
        in_specs=[_packed_spec()] * 8 +
                 [_full_spec((512, 512)), _full_spec((1, 512)),
                  _full_spec((512, 512)), _full_spec((1, 512))],
        out_specs=[_packed_spec()] * 4,
        out_shape=[jax.ShapeDtypeStruct((TCG, TNP, 128), f32)] * 4,
    )(*h, *[p.reshape(TCG, TNP, 128) for p in ps], W1, b1, W2, b2)

  h2 = gin_layer(h1, bd["c2_W1"], b["c2_b1"], bd["c2_W2"], b["c2_b2"])

  ps3 = _sc_agg4(*[hk.reshape(N_NODES, CW) for hk in h2], idx3, zero)
  outP = pl.pallas_call(
      _tc_final,
      grid=(grid,),
      in_specs=[_packed_spec()] * 8 +
               [_full_spec((512, 512)), _full_spec((1, 512)),
                _full_spec((512, 512)), _full_spec((1, 512)),
                _full_spec((512, 512)), _full_spec((1, 512)),
                _full_spec((512, 8 * 28)), _full_spec((1, 8 * 28))],
      out_specs=pl.BlockSpec((1, TNP, 8 * 28), lambda i: (i, 0, 0)),
      out_shape=jax.ShapeDtypeStruct((TCG, TNP, 8 * 28), f32),
  )(*h2, *[p.reshape(TCG, TNP, 128) for p in ps3],
    bd["c3_W1"], b["c3_b1"], bd["c3_W2"], b["c3_b2"],
    bd["f_W1"], b["f_b1"], bd["f_W2"], b["f_b2"])
  return outP.reshape(N_NODES, 28)


# speedup vs baseline: 1.8559x; 1.8559x over previous
import jax
import jax.numpy as jnp
from jax import lax
from jax.experimental import pallas as pl
from jax.experimental.pallas import tpu as pltpu
from jax.experimental.pallas import tpu_sc as plsc

N_NODES = 100000
N_EDGES = 1600000
CW = 16
GRP = 125
GPS = 5
ROWS = N_EDGES // GRP
NC, NS = 2, 16
ROWS_PER_SC = ROWS // NC
ROWS_PER_TILE = ROWS_PER_SC // NS
STEPS = ROWS_PER_TILE // GPS
NPT = N_NODES // 10


def _make_sc_agg(nchunks):
  edge_split = nchunks == 1
  steps = STEPS if edge_split else 2 * STEPS
  rows_per_tile = ROWS_PER_TILE if edge_split else ROWS // NS

  def body(*refs):
    feats = refs[:nchunks]
    idx3, zero = refs[nchunks:nchunks + 2]
    outs = refs[nchunks + 2:2 * nchunks + 2]
    (acc, srcv0, dstv0, rows0, sem0,
     srcv1, dstv1, rows1, sem1) = refs[2 * nchunks + 2:]

    c = lax.axis_index("c")
    s = lax.axis_index("s")
    if edge_split:
      base = (c * NS + s) * rows_per_tile
    else:
      base = s * rows_per_tile

    def load_idx(r0, srcv, dstv):
      pltpu.sync_copy(idx3.at[0, pl.ds(r0, GPS)], srcv)
      pltpu.sync_copy(idx3.at[1, pl.ds(r0, GPS)], dstv)

    def chunk_pass(k):
      feat = feats[k]
      out = outs[k]

      def fire(srcv, rows, sem):
        for j in range(GPS):
          pltpu.async_copy(feat.at[srcv.at[j]], rows.at[j], sem)

      def drain(srcv, rows, sem):
        for j in range(GPS):
          pltpu.make_async_copy(feat.at[srcv.at[j]], rows.at[j], sem).wait()

      def scatter(dstv, rows):
        for j in range(GPS):
          pltpu.sync_copy(rows.at[j], acc.at[dstv.at[j]], add=True)

      @pl.when(s < 10)
      def _():
        pltpu.sync_copy(zero, acc.at[pl.ds(pl.multiple_of(s * NPT, 8), NPT)])
      plsc.subcore_barrier()

      load_idx(base, srcv0, dstv0)
      fire(srcv0, rows0, sem0)

      def pair(t, _):
        r0 = base + 2 * t * GPS
        load_idx(r0 + GPS, srcv1, dstv1)
        fire(srcv1, rows1, sem1)
        drain(srcv0, rows0, sem0)
        scatter(dstv0, rows0)
        load_idx(r0 + 2 * GPS, srcv0, dstv0)
        fire(srcv0, rows0, sem0)
        drain(srcv1, rows1, sem1)
        scatter(dstv1, rows1)
        return _

      lax.fori_loop(0, steps // 2 - 1, pair, 0)
      load_idx(base + (steps - 1) * GPS, srcv1, dstv1)
      fire(srcv1, rows1, sem1)
      drain(srcv0, rows0, sem0)
      scatter(dstv0, rows0)
      drain(srcv1, rows1, sem1)
      scatter(dstv1, rows1)
      plsc.subcore_barrier()

      @pl.when(s < 10)
      def _():
        row0 = pl.multiple_of(s * NPT, 8)
        if edge_split:
          dst = out.at[c, pl.ds(row0, NPT)]
        else:
          dst = out.at[pl.ds(row0, NPT)]
        pltpu.sync_copy(acc.at[pl.ds(row0, NPT)], dst)
      plsc.subcore_barrier()

    if edge_split:
      chunk_pass(0)
    else:
      @pl.when(c == 0)
      def _():
        chunk_pass(0)
        chunk_pass(1)

      @pl.when(c == 1)
      def _():
        chunk_pass(2)
        chunk_pass(3)

  out_shape = ((NC, N_NODES, CW) if edge_split else (N_NODES, CW))
  mesh = plsc.VectorSubcoreMesh(core_axis_name="c", subcore_axis_name="s")
  return pl.kernel(
      body,
      out_type=[jax.ShapeDtypeStruct(out_shape, jnp.float32)
                for _ in range(nchunks)],
      mesh=mesh,
      compiler_params=pltpu.CompilerParams(use_tc_tiling_on_sc=False),
      scratch_types=[
          pltpu.VMEM_SHARED((N_NODES, CW), jnp.float32),
          pltpu.VMEM((GPS, GRP), jnp.int32),
          pltpu.VMEM((GPS, GRP), jnp.int32),
          pltpu.VMEM((GPS, GRP, CW), jnp.float32),
          pltpu.SemaphoreType.DMA,
          pltpu.VMEM((GPS, GRP), jnp.int32),
          pltpu.VMEM((GPS, GRP), jnp.int32),
          pltpu.VMEM((GPS, GRP, CW), jnp.float32),
          pltpu.SemaphoreType.DMA,
      ],
  )


_sc_agg1 = _make_sc_agg(1)
_sc_agg4 = _make_sc_agg(4)

NP_ROWS = N_NODES // 8
TCG = 10
TNP = NP_ROWS // TCG


def _bd(W):
  W4 = W.reshape(4, CW, 4, CW)
  return jnp.einsum('kfpg,ab->kafpbg', W4, jnp.eye(8, dtype=W.dtype)
                    ).reshape(512, 512)


def _bd_in(W):
  W3 = W.reshape(CW, 4, CW)
  return jnp.einsum('fpg,ab->afpbg', W3, jnp.eye(8, dtype=W.dtype)
                    ).reshape(128, 512)


def _bd_out(W):
  out = W.shape[1]
  W3 = W.reshape(4, CW, out)
  return jnp.einsum('pgo,ba->pbgao', W3, jnp.eye(8, dtype=W.dtype)
                    ).reshape(512, 8 * out)


def _bvec(bv):
  return jnp.tile(bv.reshape(4, 1, CW), (1, 8, 1)).reshape(1, 512)


def _mlp(h, W1, b1, W2, b2):
  t = jnp.maximum(jnp.dot(h, W1, preferred_element_type=jnp.float32) + b1, 0.0)
  return jnp.dot(t, W2, preferred_element_type=jnp.float32) + b2


def _split_out(y, outs):
  for k, o in enumerate(outs):
    o[...] = y[:, 128 * k:128 * (k + 1)].reshape(1, TNP, 128)


def _tc_mlp1(xp_ref, p_ref, W1_ref, b1_ref, W2_ref, b2_ref, o0, o1, o2, o3):
  h = xp_ref[0] + p_ref[0, 0] + p_ref[1, 0]
  y = _mlp(h, W1_ref[...], b1_ref[...], W2_ref[...], b2_ref[...])
  _split_out(y, (o0, o1, o2, o3))


def _tc_mlp(h0, h1, h2, h3, p0, p1, p2, p3,
            W1_ref, b1_ref, W2_ref, b2_ref, o0, o1, o2, o3):
  h = jnp.concatenate([h0[0] + p0[0], h1[0] + p1[0],
                       h2[0] + p2[0], h3[0] + p3[0]], axis=1)
  y = _mlp(h, W1_ref[...], b1_ref[...], W2_ref[...], b2_ref[...])
  _split_out(y, (o0, o1, o2, o3))


def _tc_final(h0, h1, h2, h3, p0, p1, p2, p3,
              W1_ref, b1_ref, W2_ref, b2_ref,
              fW1_ref, fb1_ref, fW2_ref, fb2_ref, out_ref):
  h = jnp.concatenate([h0[0] + p0[0], h1[0] + p1[0],
                       h2[0] + p2[0], h3[0] + p3[0]], axis=1)
  y = _mlp(h, W1_ref[...], b1_ref[...], W2_ref[...], b2_ref[...])
  t = jnp.maximum(
      jnp.dot(y, fW1_ref[...], preferred_element_type=jnp.float32)
      + fb1_ref[...], 0.0)
  o = (jnp.dot(t, fW2_ref[...], preferred_element_type=jnp.float32)
       + fb2_ref[...])
  out_ref[...] = o.reshape(1, TNP, 8 * 28)


def _packed_spec():
  return pl.BlockSpec((1, TNP, 128), lambda i: (i, 0, 0))


def _partial_spec():
  return pl.BlockSpec((NC, 1, TNP, 128), lambda i: (0, i, 0, 0))


def _full_spec(shape):
  nd = len(shape)
  return pl.BlockSpec(shape, lambda i, _nd=nd: (0,) * _nd)


@jax.jit
def kernel(x, edge_index, c1_W1, c1_b1, c1_W2, c1_b2, c2_W1, c2_b1, c2_W2,
           c2_b2, c3_W1, c3_b1, c3_W2, c3_b2, f_W1, f_b1, f_W2, f_b2):
  grid = TCG
  f32 = jnp.float32

  xp = jnp.pad(x, ((0, 0), (0, CW - x.shape[1])))
  xpP = xp.reshape(TCG, TNP, 128)
  idx3 = edge_index.reshape(2, ROWS, GRP)
  zero = jnp.zeros((NPT, CW), f32)
  W1p = jnp.pad(c1_W1, ((0, CW - c1_W1.shape[0]), (0, 0)))
  bd = {"c1_W1": _bd_in(W1p), "c1_W2": _bd(c1_W2),
        "c2_W1": _bd(c2_W1), "c2_W2": _bd(c2_W2),
        "c3_W1": _bd(c3_W1), "c3_W2": _bd(c3_W2),
        "f_W1": _bd(f_W1), "f_W2": _bd_out(f_W2)}
  b = {n: _bvec(v) for n, v in
       dict(c1_b1=c1_b1, c1_b2=c1_b2, c2_b1=c2_b1, c2_b2=c2_b2,
            c3_b1=c3_b1, c3_b2=c3_b2, f_b1=f_b1).items()}
  b["f_b2"] = jnp.tile(f_b2, (8,)).reshape(1, 8 * 28)

  (p1,) = _sc_agg1(xp, idx3, zero)
  h1 = pl.pallas_call(
      _tc_mlp1,
      grid=(grid,),
      in_specs=[_packed_spec(), _partial_spec(),
                _full_spec((128, 512)), _full_spec((1, 512)),
                _full_spec((512, 512)), _full_spec((1, 512))],
      out_specs=[_packed_spec()] * 4,
      out_shape=[jax.ShapeDtypeStruct((TCG, TNP, 128), f32)] * 4,
  )(xpP, p1.reshape(NC, TCG, TNP, 128), bd["c1_W1"], b["c1_b1"],
    bd["c1_W2"], b["c1_b2"])

  def gin_layer(h, W1, b1, W2, b2):
    ps = _sc_agg4(*[hk.reshape(N_NODES, CW) for hk in h], idx3, zero)
    return pl.pallas_call(
        _tc_mlp,
        grid=(grid,),
        in_specs=[_packed_spec()] * 8 +
                 [_full_spec((512, 512)), _full_spec((1, 512)),
                  _full_spec((512, 512)), _full_spec((1, 512))],
        out_specs=[_packed_spec()] * 4,
        out_shape=[jax.ShapeDtypeStruct((TCG, TNP, 128), f32)] * 4,
    )(*h, *[p.reshape(TCG, TNP, 128) for p in ps], W1, b1, W2, b2)

  h2 = gin_layer(h1, bd["c2_W1"], b["c2_b1"], bd["c2_W2"], b["c2_b2"])

  ps3 = _sc_agg4(*[hk.reshape(N_NODES, CW) for hk in h2], idx3, zero)
  outP = pl.pallas_call(
      _tc_final,
      grid=(grid,),
      in_specs=[_packed_spec()] * 8 +
               [_full_spec((512, 512)), _full_spec((1, 512)),
                _full_spec((512, 512)), _full_spec((1, 512)),
                _full_spec((512, 512)), _full_spec((1, 512)),
                _full_spec((512, 8 * 28)), _full_spec((1, 8 * 28))],
      out_specs=pl.BlockSpec((1, TNP, 8 * 28), lambda i: (i, 0, 0)),
      out_shape=jax.ShapeDtypeStruct((TCG, TNP, 8 * 28), f32),
  )(*h2, *[p.reshape(TCG, TNP, 128) for p in ps3],
    bd["c3_W1"], b["c3_b1"], bd["c3_W2"], b["c3_b2"],
    bd["f_W1"], b["f_b1"], bd["f_W2"], b["f_b2"])
  return outP.reshape(N_NODES, 28)
